# baseline (speedup 1.0000x reference)
import math

import jax
import jax.numpy as jnp
from jax.experimental import pallas as pl
from jax.experimental.pallas import tpu as pltpu

_M_TILE = 1024


def _nn_dist_kernel(e_ref, ct_ref, o_ref):
    ef = e_ref[...]
    ctf = ct_ref[...]
    eb = (-2.0 * ef).astype(jnp.bfloat16)
    ctb = ctf.astype(jnp.bfloat16)
    enorm = jnp.sum(ef * ef, axis=1, keepdims=True)
    cnorm = jnp.sum(ctf * ctf, axis=0, keepdims=True)
    prod = jax.lax.dot_general(
        eb, ctb,
        dimension_numbers=(((1,), (0,)), ((), ())),
        preferred_element_type=jnp.float32,
    )
    dmin = jnp.min(cnorm + prod, axis=1, keepdims=True)
    o_ref[...] = jnp.sqrt(enorm + dmin)


def kernel(embeds, centroids):
    B, N, D = embeds.shape
    P = centroids.shape[0]
    M = B * N
    e2 = embeds.reshape(M, D)
    ct = centroids.T
    n_tiles = M // _M_TILE

    out = pl.pallas_call(
        _nn_dist_kernel,
        grid=(n_tiles,),
        in_specs=[
            pl.BlockSpec((_M_TILE, D), lambda i: (i, 0)),
            pl.BlockSpec((D, P), lambda i: (0, 0)),
        ],
        out_specs=pl.BlockSpec((_M_TILE, 1), lambda i: (i, 0)),
        out_shape=jax.ShapeDtypeStruct((M, 1), jnp.float32),
    )(e2, ct)

    h = int(math.sqrt(N))
    score = out.reshape(B, 1, h, h)
    loss = jnp.zeros(())
    return (loss, score)


# speedup vs baseline: 1.2917x; 1.2917x over previous
import math

import jax
import jax.numpy as jnp
from jax.experimental import pallas as pl

_M_TILE = 1024


def _nn_dist_kernel(e_ref, ct_ref, o_ref):
    ef = e_ref[...]
    ctf = ct_ref[...]
    eb = ef.astype(jnp.bfloat16)
    ctb = ctf.astype(jnp.bfloat16)
    enorm = jnp.sum(ef * ef, axis=1)
    cnorm = 0.25 * jnp.sum(ctf * ctf, axis=0, keepdims=True)
    prod = jax.lax.dot_general(
        eb, ctb,
        dimension_numbers=(((1,), (0,)), ((), ())),
        preferred_element_type=jnp.float32,
    )
    dmin = jnp.min(cnorm + prod, axis=1)
    o_ref[...] = jnp.sqrt(enorm + dmin).reshape(o_ref.shape)


def kernel(embeds, centroids):
    B, N, D = embeds.shape
    P = centroids.shape[0]
    M = B * N
    e2 = embeds.reshape(M, D)
    ct = (-2.0 * centroids).T
    n_tiles = M // _M_TILE
    rows_out = _M_TILE // 128

    out = pl.pallas_call(
        _nn_dist_kernel,
        grid=(n_tiles,),
        in_specs=[
            pl.BlockSpec((_M_TILE, D), lambda i: (i, 0)),
            pl.BlockSpec((D, P), lambda i: (0, 0)),
        ],
        out_specs=pl.BlockSpec((rows_out, 128), lambda i: (i, 0)),
        out_shape=jax.ShapeDtypeStruct((n_tiles * rows_out, 128), jnp.float32),
    )(e2, ct)

    h = int(math.sqrt(N))
    score = out.reshape(B, 1, h, h)
    loss = jnp.zeros(())
    return (loss, score)
